# baseline (speedup 1.0000x reference)
import functools

import jax
import jax.numpy as jnp
from jax import lax
from jax.experimental import pallas as pl
from jax.experimental.pallas import tpu as pltpu
from jax.experimental.pallas import tpu_sc as plsc

D = 128
L = 16
ND = D // L


@functools.lru_cache(maxsize=None)
def _make(E, V):
    info = plsc.get_sparse_core_info()
    NC, NS = info.num_cores, info.num_subcores
    NW = NC * NS
    assert E % NW == 0
    EPW = E // NW
    C = 80
    NB = 2
    assert EPW % C == 0 and C % L == 0
    NCH = EPW // C
    NG = C // L
    NP = NCH // NB
    REM = NCH - NP * NB

    mesh = plsc.VectorSubcoreMesh(core_axis_name="c", subcore_axis_name="s")

    @functools.partial(
        pl.kernel,
        mesh=mesh,
        compiler_params=pltpu.CompilerParams(
            needs_layout_passes=False, use_tc_tiling_on_sc=False
        ),
        out_type=jax.ShapeDtypeStruct((E,), jnp.float32),
        scratch_types=[
            pltpu.VMEM((EPW,), jnp.int32),
            pltpu.VMEM((EPW,), jnp.int32),
            pltpu.VMEM((EPW,), jnp.float32),
            pltpu.VMEM((C * L,), jnp.float32),
        ]
        + [pltpu.VMEM((C, D // 2), jnp.int32) for _ in range(2 * NB)]
        + [pltpu.SemaphoreType.DMA for _ in range(2 * NB)],
    )
    def k(zsrc, zdst, eidx, out, ia0, ia1, outa, tbuf, *bufs_sems):
        srows = bufs_sems[0:NB]
        drows = bufs_sems[NB : 2 * NB]
        sems = bufs_sems[2 * NB : 4 * NB]
        wid = lax.axis_index("s") * NC + lax.axis_index("c")
        base_w = wid * EPW
        lane = lax.iota(jnp.int32, L)
        idx_t = lane * L

        cp0 = pltpu.async_copy(eidx.at[0, pl.ds(base_w, EPW)], ia0, sems[0])
        cp1 = pltpu.async_copy(eidx.at[1, pl.ds(base_w, EPW)], ia1, sems[1])
        cp0.wait()
        cp1.wait()

        @plsc.parallel_loop(0, EPW // L)
        def zero_body(i):
            outa[pl.ds(i * L, L)] = jnp.zeros((L,), jnp.float32)

        def issue(ci, b):
            off = ci * C
            pltpu.async_copy(zsrc.at[ia0.at[pl.ds(off, C)]], srows[b], sems[2 * b])
            pltpu.async_copy(zdst.at[ia1.at[pl.ds(off, C)]], drows[b], sems[2 * b + 1])

        def wait(ci, b):
            off = ci * C
            pltpu.make_async_copy(
                zsrc.at[ia0.at[pl.ds(off, C)]], srows[b], sems[2 * b]
            ).wait()
            pltpu.make_async_copy(
                zdst.at[ia1.at[pl.ds(off, C)]], drows[b], sems[2 * b + 1]
            ).wait()

        def compute(ci, b):
            sr, dr = srows[b], drows[b]

            @plsc.parallel_loop(0, C)
            def edge_body(e):
                parts = []
                for q in range(D // (2 * L)):
                    s2 = plsc.bitcast(sr[e, pl.ds(q * L, L)], jnp.bfloat16)
                    d2 = plsc.bitcast(dr[e, pl.ds(q * L, L)], jnp.bfloat16)
                    pa, pb = plsc.unpack(
                        s2 * d2,
                        format=plsc.PackFormat.INTERLEAVED,
                        preferred_element_type=jnp.float32,
                    )
                    parts += [pa, pb]
                while len(parts) > 1:
                    parts = [a + b2 for a, b2 in zip(parts[::2], parts[1::2])]
                plsc.addupdate_scatter(
                    outa, [jnp.zeros((L,), jnp.int32) + (ci * C + e)], parts[0]
                )

        for j in range(NB - 1):
            issue(j, j)

        def pair_body(p, carry):
            c0 = p * NB
            for b in range(NB):
                ci = c0 + b

                @pl.when(ci + NB - 1 < NCH)
                def _():
                    issue(ci + NB - 1, (b + NB - 1) % NB)

                wait(ci, b)
                compute(ci, b)
            return carry

        lax.fori_loop(0, NP, pair_body, 0)
        for j in range(REM):
            ci = NP * NB + j
            wait(ci, j)
            compute(ci, j)

        pltpu.sync_copy(outa, out.at[pl.ds(base_w, EPW)])

    return k


def kernel(z_src, z_dst, edge_label_index):
    E = edge_label_index.shape[1]
    V, Dd = z_src.shape
    H = Dd // 2

    def rne(x):
        return (x + 0x7FFF + ((x >> 16) & 1)) >> 16

    def as_words(z):
        u = jax.lax.bitcast_convert_type(z, jnp.uint32)
        w = (rne(u[:, H:]) << 16) | rne(u[:, :H])
        return jax.lax.bitcast_convert_type(w, jnp.int32)

    return _make(E, V)(as_words(z_src), as_words(z_dst), edge_label_index)


# speedup vs baseline: 1.9484x; 1.9484x over previous
import functools

import jax
import jax.numpy as jnp
from jax import lax
from jax.experimental import pallas as pl
from jax.experimental.pallas import tpu as pltpu
from jax.experimental.pallas import tpu_sc as plsc

D = 128
L = 16
ND = D // L


@functools.lru_cache(maxsize=None)
def _make(E, V):
    info = plsc.get_sparse_core_info()
    NC, NS = info.num_cores, info.num_subcores
    NW = NC * NS
    assert E % NW == 0
    EPW = E // NW
    C = 80
    NB = 2
    assert EPW % C == 0 and C % L == 0
    NCH = EPW // C
    NG = C // L
    NP = NCH // NB
    REM = NCH - NP * NB

    mesh = plsc.VectorSubcoreMesh(core_axis_name="c", subcore_axis_name="s")

    @functools.partial(
        pl.kernel,
        mesh=mesh,
        compiler_params=pltpu.CompilerParams(
            needs_layout_passes=False, use_tc_tiling_on_sc=False
        ),
        out_type=jax.ShapeDtypeStruct((E,), jnp.float32),
        scratch_types=[
            pltpu.VMEM((EPW,), jnp.int32),
            pltpu.VMEM((EPW,), jnp.int32),
            pltpu.VMEM((EPW,), jnp.float32),
            pltpu.VMEM((C * L,), jnp.float32),
        ]
        + [pltpu.VMEM((C, D // 2), jnp.int32) for _ in range(2 * NB)]
        + [pltpu.SemaphoreType.DMA for _ in range(2 * NB)],
    )
    def k(zsrc, zdst, eidx, out, ia0, ia1, outa, tbuf, *bufs_sems):
        srows = bufs_sems[0:NB]
        drows = bufs_sems[NB : 2 * NB]
        sems = bufs_sems[2 * NB : 4 * NB]
        wid = lax.axis_index("s") * NC + lax.axis_index("c")
        base_w = wid * EPW
        lane = lax.iota(jnp.int32, L)
        idx_t = lane * L

        cp0 = pltpu.async_copy(eidx.at[0, pl.ds(base_w, EPW)], ia0, sems[0])
        cp1 = pltpu.async_copy(eidx.at[1, pl.ds(base_w, EPW)], ia1, sems[1])
        cp0.wait()
        cp1.wait()

        def issue(ci, b):
            off = ci * C
            pltpu.async_copy(zsrc.at[ia0.at[pl.ds(off, C)]], srows[b], sems[2 * b])
            pltpu.async_copy(zdst.at[ia1.at[pl.ds(off, C)]], drows[b], sems[2 * b + 1])

        def wait(ci, b):
            off = ci * C
            pltpu.make_async_copy(
                zsrc.at[ia0.at[pl.ds(off, C)]], srows[b], sems[2 * b]
            ).wait()
            pltpu.make_async_copy(
                zdst.at[ia1.at[pl.ds(off, C)]], drows[b], sems[2 * b + 1]
            ).wait()

        def compute(ci, b):
            sr, dr = srows[b], drows[b]

            @plsc.parallel_loop(0, C, unroll=2)
            def edge_body(e):
                parts = []
                for q in range(D // (2 * L)):
                    s2 = plsc.bitcast(sr[e, pl.ds(q * L, L)], jnp.bfloat16)
                    d2 = plsc.bitcast(dr[e, pl.ds(q * L, L)], jnp.bfloat16)
                    pa, pb = plsc.unpack(
                        s2 * d2,
                        format=plsc.PackFormat.INTERLEAVED,
                        preferred_element_type=jnp.float32,
                    )
                    parts += [pa, pb]
                while len(parts) > 1:
                    parts = [a + b2 for a, b2 in zip(parts[::2], parts[1::2])]
                tbuf[pl.ds(e * L, L)] = parts[0]

            @plsc.parallel_loop(0, NG)
            def red_body(g):
                tb = g * L * L
                cols = [plsc.load_gather(tbuf, [idx_t + (tb + c)]) for c in range(L)]
                while len(cols) > 1:
                    cols = [a + b2 for a, b2 in zip(cols[::2], cols[1::2])]
                outa[pl.ds(ci * C + g * L, L)] = cols[0]

        for j in range(NB - 1):
            issue(j, j)

        def pair_body(p, carry):
            c0 = p * NB
            for b in range(NB):
                ci = c0 + b

                @pl.when(ci + NB - 1 < NCH)
                def _():
                    issue(ci + NB - 1, (b + NB - 1) % NB)

                wait(ci, b)
                compute(ci, b)
            return carry

        lax.fori_loop(0, NP, pair_body, 0)
        for j in range(REM):
            ci = NP * NB + j
            wait(ci, j)
            compute(ci, j)

        pltpu.sync_copy(outa, out.at[pl.ds(base_w, EPW)])

    return k


def kernel(z_src, z_dst, edge_label_index):
    E = edge_label_index.shape[1]
    V, Dd = z_src.shape
    H = Dd // 2

    def rne(x):
        return (x + 0x7FFF + ((x >> 16) & 1)) >> 16

    def as_words(z):
        u = jax.lax.bitcast_convert_type(z, jnp.uint32)
        w = (rne(u[:, H:]) << 16) | rne(u[:, :H])
        return jax.lax.bitcast_convert_type(w, jnp.int32)

    return _make(E, V)(as_words(z_src), as_words(z_dst), edge_label_index)


# speedup vs baseline: 1.9493x; 1.0005x over previous
import functools

import jax
import jax.numpy as jnp
from jax import lax
from jax.experimental import pallas as pl
from jax.experimental.pallas import tpu as pltpu
from jax.experimental.pallas import tpu_sc as plsc

D = 128
L = 16
ND = D // L


@functools.lru_cache(maxsize=None)
def _make(E, V):
    info = plsc.get_sparse_core_info()
    NC, NS = info.num_cores, info.num_subcores
    NW = NC * NS
    assert E % NW == 0
    EPW = E // NW
    C = 80
    NB = 2
    assert EPW % C == 0 and C % L == 0
    NCH = EPW // C
    NG = C // L
    NP = NCH // NB
    REM = NCH - NP * NB

    mesh = plsc.VectorSubcoreMesh(core_axis_name="c", subcore_axis_name="s")

    @functools.partial(
        pl.kernel,
        mesh=mesh,
        compiler_params=pltpu.CompilerParams(
            needs_layout_passes=False, use_tc_tiling_on_sc=False
        ),
        out_type=jax.ShapeDtypeStruct((E,), jnp.float32),
        scratch_types=[
            pltpu.VMEM((EPW,), jnp.int32),
            pltpu.VMEM((EPW,), jnp.int32),
            pltpu.VMEM((EPW,), jnp.float32),
            pltpu.VMEM((C * L,), jnp.float32),
        ]
        + [pltpu.VMEM((C, D // 2), jnp.int32) for _ in range(2 * NB)]
        + [pltpu.SemaphoreType.DMA for _ in range(2 * NB)],
    )
    def k(zsrc, zdst, eidx, out, ia0, ia1, outa, tbuf, *bufs_sems):
        srows = bufs_sems[0:NB]
        drows = bufs_sems[NB : 2 * NB]
        sems = bufs_sems[2 * NB : 4 * NB]
        wid = lax.axis_index("s") * NC + lax.axis_index("c")
        base_w = wid * EPW
        lane = lax.iota(jnp.int32, L)
        idx_t = lane * L

        cp0 = pltpu.async_copy(eidx.at[0, pl.ds(base_w, EPW)], ia0, sems[0])
        cp1 = pltpu.async_copy(eidx.at[1, pl.ds(base_w, EPW)], ia1, sems[1])
        cp0.wait()
        cp1.wait()

        def issue(ci, b):
            off = ci * C
            pltpu.async_copy(zsrc.at[ia0.at[pl.ds(off, C)]], srows[b], sems[2 * b])
            pltpu.async_copy(zdst.at[ia1.at[pl.ds(off, C)]], drows[b], sems[2 * b + 1])

        def wait(ci, b):
            off = ci * C
            pltpu.make_async_copy(
                zsrc.at[ia0.at[pl.ds(off, C)]], srows[b], sems[2 * b]
            ).wait()
            pltpu.make_async_copy(
                zdst.at[ia1.at[pl.ds(off, C)]], drows[b], sems[2 * b + 1]
            ).wait()

        def compute(ci, b):
            sr, dr = srows[b], drows[b]

            @plsc.parallel_loop(0, C, unroll=4)
            def edge_body(e):
                parts = []
                for q in range(D // (2 * L)):
                    s2 = plsc.bitcast(sr[e, pl.ds(q * L, L)], jnp.bfloat16)
                    d2 = plsc.bitcast(dr[e, pl.ds(q * L, L)], jnp.bfloat16)
                    pa, pb = plsc.unpack(
                        s2 * d2,
                        format=plsc.PackFormat.INTERLEAVED,
                        preferred_element_type=jnp.float32,
                    )
                    parts += [pa, pb]
                while len(parts) > 1:
                    parts = [a + b2 for a, b2 in zip(parts[::2], parts[1::2])]
                tbuf[pl.ds(e * L, L)] = parts[0]

            @plsc.parallel_loop(0, NG)
            def red_body(g):
                tb = g * L * L
                cols = [plsc.load_gather(tbuf, [idx_t + (tb + c)]) for c in range(L)]
                while len(cols) > 1:
                    cols = [a + b2 for a, b2 in zip(cols[::2], cols[1::2])]
                outa[pl.ds(ci * C + g * L, L)] = cols[0]

        for j in range(NB - 1):
            issue(j, j)

        def pair_body(p, carry):
            c0 = p * NB
            for b in range(NB):
                ci = c0 + b

                @pl.when(ci + NB - 1 < NCH)
                def _():
                    issue(ci + NB - 1, (b + NB - 1) % NB)

                wait(ci, b)
                compute(ci, b)
            return carry

        lax.fori_loop(0, NP, pair_body, 0)
        for j in range(REM):
            ci = NP * NB + j
            wait(ci, j)
            compute(ci, j)

        pltpu.sync_copy(outa, out.at[pl.ds(base_w, EPW)])

    return k


def kernel(z_src, z_dst, edge_label_index):
    E = edge_label_index.shape[1]
    V, Dd = z_src.shape
    H = Dd // 2

    def rne(x):
        return (x + 0x7FFF + ((x >> 16) & 1)) >> 16

    def as_words(z):
        u = jax.lax.bitcast_convert_type(z, jnp.uint32)
        w = (rne(u[:, H:]) << 16) | rne(u[:, :H])
        return jax.lax.bitcast_convert_type(w, jnp.int32)

    return _make(E, V)(as_words(z_src), as_words(z_dst), edge_label_index)
